# baseline (speedup 1.0000x reference)
import jax
import jax.numpy as jnp
import numpy as np
from jax.experimental import pallas as pl

_PA = np.zeros((4, 128), np.float32)
_PB = np.zeros((4, 128), np.float32)
_PA[0, :64], _PA[1, :64], _PA[2, 64:], _PA[3, 64:] = -1.0, 1.0, -1.0, 1.0
_PB[0, :64], _PB[1, :64], _PB[2, 64:], _PB[3, 64:] = 1.0, 1.0, 1.0, 1.0


def _phi_body(d_ref, pa_ref, pb_ref, s_ref, t_ref, o_ref):
    th = t_ref[0, 0]
    c = 0.5 * th

    d = c * d_ref[...]
    dn = (((1,), (0,)), ((), ()))
    ta = jax.lax.dot_general(d, pa_ref[...], dn,
                             preferred_element_type=jnp.float32)
    tb = jax.lax.dot_general(d, pb_ref[...], dn,
                             preferred_element_type=jnp.float32)

    ts64 = th * s_ref[...]
    ts = jnp.concatenate([ts64, ts64], axis=1)

    w = jnp.abs(ts - tb) - ta
    o_ref[...] = 1.0 / (1.0 + jnp.exp(w))


def kernel(diagrams, samples, theta):
    n, p, _ = diagrams.shape
    s = samples.shape[0]
    rows = n * p // 2
    blk = 1024

    d4 = diagrams.reshape(rows, 4)
    s2 = samples.reshape(1, s)
    t2 = jnp.reshape(theta, (1, 1))

    out = pl.pallas_call(
        _phi_body,
        grid=(rows // blk,),
        in_specs=[
            pl.BlockSpec((blk, 4), lambda i: (i, 0)),
            pl.BlockSpec((4, 128), lambda i: (0, 0)),
            pl.BlockSpec((4, 128), lambda i: (0, 0)),
            pl.BlockSpec((1, s), lambda i: (0, 0)),
            pl.BlockSpec((1, 1), lambda i: (0, 0)),
        ],
        out_specs=pl.BlockSpec((blk, 2 * s), lambda i: (i, 0)),
        out_shape=jax.ShapeDtypeStruct((rows, 2 * s), jnp.float32),
    )(d4, jnp.asarray(_PA), jnp.asarray(_PB), s2, t2)

    output = out.reshape(n, p, s)
    output_shape = jnp.array(samples.shape, dtype=jnp.int32)
    return (output, output_shape)


# speedup vs baseline: 5.6919x; 5.6919x over previous
import jax
import jax.numpy as jnp
from jax.experimental import pallas as pl


def _phi_body(v_ref, s_ref, t_ref, o_ref):
    th = t_ref[0, 0]
    c = 0.5 * th
    v = v_ref[...]
    ts_col = th * jnp.transpose(s_ref[...])
    for t in range(16):
        x = v[2 * t:2 * t + 1, :]
        y = v[2 * t + 1:2 * t + 2, :]
        ta = c * (y - x)
        tb = c * (y + x)
        w = ta - jnp.abs(ts_col - tb)
        o_ref[0, :, 128 * t:128 * (t + 1)] = jax.nn.sigmoid(w)


def kernel(diagrams, samples, theta):
    n, p, _ = diagrams.shape
    s = samples.shape[0]

    v = diagrams.reshape(n, p // 128, 128, 2).transpose(0, 1, 3, 2)
    v = v.reshape(n * (p // 128) * 2, 128)
    s2 = samples.reshape(1, s)
    t2 = jnp.reshape(theta, (1, 1))

    out3 = pl.pallas_call(
        _phi_body,
        grid=(n,),
        in_specs=[
            pl.BlockSpec(((p // 128) * 2, 128), lambda i: (i, 0)),
            pl.BlockSpec((1, s), lambda i: (0, 0)),
            pl.BlockSpec((1, 1), lambda i: (0, 0)),
        ],
        out_specs=pl.BlockSpec((1, s, p), lambda i: (i, 0, 0)),
        out_shape=jax.ShapeDtypeStruct((n, s, p), jnp.float32),
    )(v, s2, t2)

    output = out3.transpose(0, 2, 1)
    output_shape = jnp.array(samples.shape, dtype=jnp.int32)
    return (output, output_shape)


# speedup vs baseline: 8.0447x; 1.4134x over previous
import jax
import jax.numpy as jnp
from jax.experimental import pallas as pl


_G = 2


def _phi_body(v_ref, s_ref, t_ref, o_ref):
    th = t_ref[0, 0]
    c = 0.5 * th
    v = v_ref[...]
    ts_col = th * jnp.transpose(s_ref[...])
    for g in range(_G):
        for t in range(16):
            r = 32 * g + 2 * t
            x = v[r:r + 1, :]
            y = v[r + 1:r + 2, :]
            ta = c * (y - x)
            tb = c * (y + x)
            w = ta - jnp.abs(ts_col - tb)
            o_ref[g, :, 128 * t:128 * (t + 1)] = jax.nn.sigmoid(w)


def kernel(diagrams, samples, theta):
    n, p, _ = diagrams.shape
    s = samples.shape[0]

    v = diagrams.reshape(n, p // 128, 128, 2).transpose(0, 1, 3, 2)
    v = v.reshape(n * (p // 128) * 2, 128)
    s2 = samples.reshape(1, s)
    t2 = jnp.reshape(theta, (1, 1))

    out3 = pl.pallas_call(
        _phi_body,
        grid=(n // _G,),
        in_specs=[
            pl.BlockSpec(((p // 128) * 2 * _G, 128), lambda i: (i, 0)),
            pl.BlockSpec((1, s), lambda i: (0, 0)),
            pl.BlockSpec((1, 1), lambda i: (0, 0)),
        ],
        out_specs=pl.BlockSpec((_G, s, p), lambda i: (i, 0, 0)),
        out_shape=jax.ShapeDtypeStruct((n, s, p), jnp.float32),
    )(v, s2, t2)

    output = out3.transpose(0, 2, 1)
    output_shape = jnp.array(samples.shape, dtype=jnp.int32)
    return (output, output_shape)


# speedup vs baseline: 9.3477x; 1.1620x over previous
import jax
import jax.numpy as jnp
from jax.experimental import pallas as pl


_G = 4


def _phi_body(v_ref, s_ref, t_ref, o_ref):
    th = t_ref[0, 0]
    c = 0.5 * th
    v = v_ref[...]
    ts_col = th * jnp.transpose(s_ref[...])
    for g in range(_G):
        for t in range(16):
            r = 32 * g + 2 * t
            x = v[r:r + 1, :]
            y = v[r + 1:r + 2, :]
            ta = c * (y - x)
            tb = c * (y + x)
            w = ta - jnp.abs(ts_col - tb)
            o_ref[g, :, 128 * t:128 * (t + 1)] = jax.nn.sigmoid(w)


def kernel(diagrams, samples, theta):
    n, p, _ = diagrams.shape
    s = samples.shape[0]

    v = diagrams.reshape(n, p // 128, 128, 2).transpose(0, 1, 3, 2)
    v = v.reshape(n * (p // 128) * 2, 128)
    s2 = samples.reshape(1, s)
    t2 = jnp.reshape(theta, (1, 1))

    out3 = pl.pallas_call(
        _phi_body,
        grid=(n // _G,),
        in_specs=[
            pl.BlockSpec(((p // 128) * 2 * _G, 128), lambda i: (i, 0)),
            pl.BlockSpec((1, s), lambda i: (0, 0)),
            pl.BlockSpec((1, 1), lambda i: (0, 0)),
        ],
        out_specs=pl.BlockSpec((_G, s, p), lambda i: (i, 0, 0)),
        out_shape=jax.ShapeDtypeStruct((n, s, p), jnp.float32),
    )(v, s2, t2)

    output = out3.transpose(0, 2, 1)
    output_shape = jnp.array(samples.shape, dtype=jnp.int32)
    return (output, output_shape)


# speedup vs baseline: 10.1517x; 1.0860x over previous
import jax
import jax.numpy as jnp
from jax.experimental import pallas as pl


_G = 8


def _phi_body(v_ref, s_ref, t_ref, o_ref):
    th = t_ref[0, 0]
    c = 0.5 * th
    v = v_ref[...]
    ts_col = th * jnp.transpose(s_ref[...])
    for g in range(_G):
        for t in range(16):
            r = 32 * g + 2 * t
            x = v[r:r + 1, :]
            y = v[r + 1:r + 2, :]
            ta = c * (y - x)
            tb = c * (y + x)
            w = ta - jnp.abs(ts_col - tb)
            o_ref[g, :, 128 * t:128 * (t + 1)] = jax.nn.sigmoid(w)


def kernel(diagrams, samples, theta):
    n, p, _ = diagrams.shape
    s = samples.shape[0]

    v = diagrams.reshape(n, p // 128, 128, 2).transpose(0, 1, 3, 2)
    v = v.reshape(n * (p // 128) * 2, 128)
    s2 = samples.reshape(1, s)
    t2 = jnp.reshape(theta, (1, 1))

    out3 = pl.pallas_call(
        _phi_body,
        grid=(n // _G,),
        in_specs=[
            pl.BlockSpec(((p // 128) * 2 * _G, 128), lambda i: (i, 0)),
            pl.BlockSpec((1, s), lambda i: (0, 0)),
            pl.BlockSpec((1, 1), lambda i: (0, 0)),
        ],
        out_specs=pl.BlockSpec((_G, s, p), lambda i: (i, 0, 0)),
        out_shape=jax.ShapeDtypeStruct((n, s, p), jnp.float32),
    )(v, s2, t2)

    output = out3.transpose(0, 2, 1)
    output_shape = jnp.array(samples.shape, dtype=jnp.int32)
    return (output, output_shape)
